# baseline (speedup 1.0000x reference)
import functools

import jax
import jax.numpy as jnp
from jax import lax
from jax.experimental import pallas as pl
from jax.experimental.pallas import tpu as pltpu
from jax.experimental.pallas import tpu_sc as plsc

EMBED_DIM = 384
D3 = EMBED_DIM // 3
N = 48
BB = 128
NTILES = 32
LANES = 16
GROUPS = BB // LANES

_INF = float("inf")


def _sc_body(xs_hbm, ys_hbm, zs_hbm, cd_hbm, m1_hbm, m2_hbm, m3_hbm,
             xv, yv, zv, cdv, m1v, m2v, m3v):
    wid = lax.axis_index("s") * 2 + lax.axis_index("c")
    pltpu.sync_copy(xs_hbm.at[wid], xv)
    pltpu.sync_copy(ys_hbm.at[wid], yv)
    pltpu.sync_copy(zs_hbm.at[wid], zv)

    for g in range(GROUPS):
        lane = pl.ds(g * LANES, LANES)
        zero = jnp.zeros((LANES,), jnp.float32)

        def csum(i, carry):
            sx, sy, sz = carry
            return (sx + xv[i, lane], sy + yv[i, lane], sz + zv[i, lane])

        sx, sy, sz = lax.fori_loop(0, N, csum, (zero, zero, zero))
        cx = sx * (1.0 / N)
        cy = sy * (1.0 / N)
        cz = sz * (1.0 / N)

        def pbody(i, carry):
            xi = xv[i, lane]
            yi = yv[i, lane]
            zi = zv[i, lane]
            dcx = xi - cx
            dcy = yi - cy
            dcz = zi - cz
            cdv[i, lane] = dcx * dcx + dcy * dcy + dcz * dcz

            def jbody(jj, ms):
                m1, m2, m3, m4 = ms
                for u in range(4):
                    j = jj * 4 + u
                    dx = xi - xv[j, lane]
                    dy = yi - yv[j, lane]
                    dz = zi - zv[j, lane]
                    d = dx * dx + dy * dy + dz * dz
                    a = jnp.maximum(m1, d)
                    m1 = jnp.minimum(m1, d)
                    b = jnp.maximum(m2, a)
                    m2 = jnp.minimum(m2, a)
                    e = jnp.maximum(m3, b)
                    m3 = jnp.minimum(m3, b)
                    m4 = jnp.minimum(m4, e)
                return (m1, m2, m3, m4)

            inf = jnp.full((LANES,), _INF, jnp.float32)
            m1, m2, m3, m4 = lax.fori_loop(0, N // 4, jbody,
                                           (inf, inf, inf, inf))
            m1v[i, lane] = m2
            m2v[i, lane] = m3
            m3v[i, lane] = m4
            return carry

        lax.fori_loop(0, N, pbody, 0)

    pltpu.sync_copy(cdv, cd_hbm.at[wid])
    pltpu.sync_copy(m1v, m1_hbm.at[wid])
    pltpu.sync_copy(m2v, m2_hbm.at[wid])
    pltpu.sync_copy(m3v, m3_hbm.at[wid])


def _sc_scalars(xs_rt, ys_rt, zs_rt):
    mesh = plsc.VectorSubcoreMesh(core_axis_name="c", subcore_axis_name="s")
    slab = jax.ShapeDtypeStruct((NTILES, N, BB), jnp.float32)
    run = functools.partial(
        pl.kernel,
        mesh=mesh,
        out_type=(slab, slab, slab, slab),
        scratch_types=[pltpu.VMEM((N, BB), jnp.float32)] * 7,
    )(_sc_body)
    return run(xs_rt, ys_rt, zs_rt)


def _tc_block(cd_ref, m1_ref, m2_ref, m3_ref, wdist_ref, bdist_ref, emb_ref,
              wdens_ref, bdens_ref, wout_ref, bout_ref, out_ref):
    cd = jnp.sqrt(cd_ref[0])
    ld = (jnp.sqrt(m1_ref[0]) + jnp.sqrt(m2_ref[0])
          + jnp.sqrt(m3_ref[0])) * (1.0 / 3.0)

    wout = wout_ref[...]
    w_lo = wout[0:D3, :]
    w_mid = wout[D3:2 * D3, :]
    w_hi = wout[2 * D3:3 * D3, :]
    v1 = jnp.dot(wdist_ref[...], w_lo, preferred_element_type=jnp.float32)
    v2 = jnp.dot(wdens_ref[...], w_hi, preferred_element_type=jnp.float32)
    cvec = (jnp.dot(bdist_ref[...], w_lo, preferred_element_type=jnp.float32)
            + jnp.dot(emb_ref[...], w_mid, preferred_element_type=jnp.float32)
            + jnp.dot(bdens_ref[...], w_hi, preferred_element_type=jnp.float32)
            + bout_ref[...])

    cd_t = jnp.transpose(cd, (1, 0))[:, :, None]
    ld_t = jnp.transpose(ld, (1, 0))[:, :, None]
    out_ref[...] = (cd_t * v1[None, :, :] + ld_t * v2[None, :, :]
                    + cvec[None, :, :])


def _tc_expand(cdsq, m1s, m2s, m3s, W_dist, b_dist, emb_row, W_dens, b_dens,
               W_out, b_out, Bv):
    slab_spec = pl.BlockSpec((1, N, BB), lambda i: (i, 0, 0))
    vec_spec = pl.BlockSpec((1, D3), lambda i: (0, 0))
    return pl.pallas_call(
        _tc_block,
        grid=(Bv // BB,),
        in_specs=[slab_spec, slab_spec, slab_spec, slab_spec,
                  vec_spec, vec_spec, vec_spec, vec_spec, vec_spec,
                  pl.BlockSpec((EMBED_DIM, EMBED_DIM), lambda i: (0, 0)),
                  pl.BlockSpec((1, EMBED_DIM), lambda i: (0, 0))],
        out_specs=pl.BlockSpec((BB, N, EMBED_DIM), lambda i: (i, 0, 0)),
        out_shape=jax.ShapeDtypeStruct((Bv, N, EMBED_DIM), jnp.float32),
    )(cdsq, m1s, m2s, m3s, W_dist, b_dist.reshape(1, D3), emb_row,
      W_dens, b_dens.reshape(1, D3), W_out, b_out.reshape(1, EMBED_DIM))


@jax.jit
def kernel(points, W_dist, b_dist, emb_count, W_dens, b_dens, W_out, b_out):
    Bv = points.shape[0]
    pts_t = jnp.transpose(points, (2, 1, 0))
    slabs = pts_t.reshape(3, N, NTILES, BB).transpose(0, 2, 1, 3)
    xs_rt, ys_rt, zs_rt = slabs[0], slabs[1], slabs[2]
    cdsq, m1s, m2s, m3s = _sc_scalars(xs_rt, ys_rt, zs_rt)
    emb_row = emb_count[N:N + 1, :]
    return _tc_expand(cdsq, m1s, m2s, m3s, W_dist, b_dist, emb_row,
                      W_dens, b_dens, W_out, b_out, Bv)


# speedup vs baseline: 1.7191x; 1.7191x over previous
import jax
import jax.numpy as jnp
from jax import lax
from jax.experimental import pallas as pl

EMBED_DIM = 384
D3 = EMBED_DIM // 3
N = 48
BB = 256

_INF = float("inf")


def _block_kernel(pts_ref, wdist_ref, bdist_ref, emb_ref,
                  wdens_ref, bdens_ref, wout_ref, bout_ref, out_ref):
    x = pts_ref[0]
    y = pts_ref[1]
    z = pts_ref[2]

    cx = jnp.mean(x, axis=0, keepdims=True)
    cy = jnp.mean(y, axis=0, keepdims=True)
    cz = jnp.mean(z, axis=0, keepdims=True)
    cd = jnp.sqrt((x - cx) ** 2 + (y - cy) ** 2 + (z - cz) ** 2)

    m1 = jnp.full((N, BB), _INF, dtype=jnp.float32)
    m2 = m1
    m3 = m1
    row = lax.broadcasted_iota(jnp.int32, (N, BB), 0)
    for j in range(N):
        dx = x - x[j:j + 1, :]
        dy = y - y[j:j + 1, :]
        dz = z - z[j:j + 1, :]
        dsq = dx * dx + dy * dy + dz * dz
        dsq = jnp.where(row == j, _INF, dsq)
        a = jnp.maximum(m1, dsq)
        m1 = jnp.minimum(m1, dsq)
        b = jnp.maximum(m2, dsq)
        m2 = jnp.minimum(m2, a)
        m3 = jnp.minimum(m3, b)
    ld = (jnp.sqrt(m1) + jnp.sqrt(m2) + jnp.sqrt(m3)) * (1.0 / 3.0)

    wout = wout_ref[...]
    w_lo = wout[0:D3, :]
    w_mid = wout[D3:2 * D3, :]
    w_hi = wout[2 * D3:3 * D3, :]
    v1 = jnp.dot(wdist_ref[...], w_lo, preferred_element_type=jnp.float32)
    v2 = jnp.dot(wdens_ref[...], w_hi, preferred_element_type=jnp.float32)
    cvec = (jnp.dot(bdist_ref[...], w_lo, preferred_element_type=jnp.float32)
            + jnp.dot(emb_ref[...], w_mid, preferred_element_type=jnp.float32)
            + jnp.dot(bdens_ref[...], w_hi, preferred_element_type=jnp.float32)
            + bout_ref[...])

    cd_t = jnp.transpose(cd, (1, 0))[:, :, None]
    ld_t = jnp.transpose(ld, (1, 0))[:, :, None]
    out_ref[...] = (cd_t * v1[None, :, :] + ld_t * v2[None, :, :]
                    + cvec[None, :, :])


def _build(interpret=False):
    def run(points, W_dist, b_dist, emb_count, W_dens, b_dens, W_out, b_out):
        Bv = points.shape[0]
        pts_t = jnp.transpose(points, (2, 1, 0))
        emb_row = emb_count[N:N + 1, :]
        return pl.pallas_call(
            _block_kernel,
            grid=(Bv // BB,),
            in_specs=[
                pl.BlockSpec((3, N, BB), lambda i: (0, 0, i)),
                pl.BlockSpec((1, D3), lambda i: (0, 0)),
                pl.BlockSpec((1, D3), lambda i: (0, 0)),
                pl.BlockSpec((1, D3), lambda i: (0, 0)),
                pl.BlockSpec((1, D3), lambda i: (0, 0)),
                pl.BlockSpec((1, D3), lambda i: (0, 0)),
                pl.BlockSpec((EMBED_DIM, EMBED_DIM), lambda i: (0, 0)),
                pl.BlockSpec((1, EMBED_DIM), lambda i: (0, 0)),
            ],
            out_specs=pl.BlockSpec((BB, N, EMBED_DIM), lambda i: (i, 0, 0)),
            out_shape=jax.ShapeDtypeStruct((Bv, N, EMBED_DIM), jnp.float32),
            interpret=interpret,
        )(pts_t, W_dist, b_dist.reshape(1, D3), emb_row,
          W_dens, b_dens.reshape(1, D3), W_out, b_out.reshape(1, EMBED_DIM))
    return run


kernel = jax.jit(_build())
